# baseline (speedup 1.0000x reference)
import functools

import jax
import jax.numpy as jnp
from jax import lax
from jax.experimental import pallas as pl
from jax.experimental.pallas import tpu as pltpu
from jax.experimental.pallas import tpu_sc as plsc

_CUTOFF = 5.0
_LOG2 = 0.6931471805599453


def _sc_interact(table, idx2d, aux, n_nbr):
    R, D = table.shape
    F = D // 3
    n_rows, CW = idx2d.shape
    E = n_rows * CW
    NW = 32
    per_w = E // NW
    n_ch = n_rows // NW
    NA = per_w // n_nbr
    A_CH = CW // n_nbr
    NV = D // 16
    KF = F // 16

    mesh = plsc.VectorSubcoreMesh(core_axis_name="c", subcore_axis_name="s")

    @functools.partial(
        pl.kernel,
        mesh=mesh,
        compiler_params=pltpu.CompilerParams(needs_layout_passes=False),
        out_type=jax.ShapeDtypeStruct((E, F), jnp.float32),
        scratch_types=[
            pltpu.VMEM((n_ch, CW), jnp.int32),
            pltpu.VMEM((NA, D), jnp.float32),
            pltpu.VMEM((per_w * 4,), jnp.float32),
            pltpu.VMEM((CW, D), jnp.float32),
            pltpu.VMEM((CW, D), jnp.float32),
            pltpu.VMEM((CW, F), jnp.float32),
            pltpu.VMEM((CW, F), jnp.float32),
            pltpu.SemaphoreType.DMA,
            pltpu.SemaphoreType.DMA,
            pltpu.SemaphoreType.DMA,
            pltpu.SemaphoreType.DMA,
        ],
    )
    def k(table_hbm, idx_hbm, aux_hbm, out_hbm, idx_v, mi_v, aux_v,
          mj0, mj1, tb0, tb1, g0, g1, p0, p1):
        wid = lax.axis_index("s") * 2 + lax.axis_index("c")
        pltpu.sync_copy(idx_hbm.at[pl.ds(wid * n_ch, n_ch)], idx_v)
        pltpu.sync_copy(table_hbm.at[pl.ds(wid * NA, NA)], mi_v)
        pltpu.sync_copy(aux_hbm.at[pl.ds(wid * per_w * 4, per_w * 4)], aux_v)
        ebase = wid * per_w

        def gather(c, mj, g):
            pltpu.async_copy(table_hbm.at[idx_v.at[c]], mj, g)

        def gwait(c, mj, g):
            pltpu.make_async_copy(table_hbm.at[idx_v.at[c]], mj, g).wait()

        def wb(c, tb, p):
            pltpu.async_copy(tb, out_hbm.at[pl.ds(ebase + c * CW, CW)], p)

        def wbwait(tb, p):
            pltpu.make_async_copy(tb, out_hbm.at[pl.ds(ebase, CW)], p).wait()

        def compute(c, mj, tb):
            for ai in range(A_CH):
                a_loc = c * A_CH + ai
                mi = [mi_v[a_loc, pl.ds(j * 16, 16)] for j in range(NV)]

                @plsc.parallel_loop(0, n_nbr, unroll=16)
                def nb_body(n):
                    e_loc = ai * n_nbr + n
                    eg = c * CW + e_loc
                    i0 = jnp.full((16,), eg * 4, jnp.int32)
                    ld = lambda col: plsc.load_gather(aux_v, [i0 + col])
                    dv0, dv1, dv2, db = ld(0), ld(1), ld(2), ld(3)
                    c3d2 = 3.0 * db * db
                    for kf in range(KF):
                        mja = mj[e_loc, pl.ds(kf * 16, 16)]
                        mjb = mj[e_loc, pl.ds(F + kf * 16, 16)]
                        mjc = mj[e_loc, pl.ds(2 * F + kf * 16, 16)]
                        pdd = (mi[kf] * mja + mi[KF + kf] * mjb
                               + mi[2 * KF + kf] * mjc)
                        qdd = dv0 * mja + dv1 * mjb + dv2 * mjc
                        tb[e_loc, pl.ds(kf * 16, 16)] = pdd * c3d2 - qdd

        gather(0, mj0, g0)

        def pair(i, _):
            c0 = 2 * i
            c1 = 2 * i + 1
            gather(c1, mj1, g1)
            gwait(c0, mj0, g0)

            @pl.when(i > 0)
            def _():
                wbwait(tb0, p0)

            compute(c0, mj0, tb0)
            wb(c0, tb0, p0)

            @pl.when(c0 + 2 < n_ch)
            def _():
                gather(c0 + 2, mj0, g0)

            gwait(c1, mj1, g1)

            @pl.when(i > 0)
            def _():
                wbwait(tb1, p1)

            compute(c1, mj1, tb1)
            wb(c1, tb1, p1)
            return 0

        lax.fori_loop(0, n_ch // 2, pair, 0)
        wbwait(tb0, p0)
        wbwait(tb1, p1)

    return k(table, idx2d, aux)


def _ssp(x):
    return jnp.maximum(x, 0.0) + jnp.log1p(jnp.exp(-jnp.abs(x))) - _LOG2


def _tc_body(t1_ref, mui_ref, fij_ref, aux_ref, dm_ref,
             we1_ref, be1_ref, we2_ref, be2_ref,
             w1_ref, b1_ref, w2_ref, b2_ref, out_ref, *, n_nbr):
    EB, F = t1_ref.shape
    TA = mui_ref.shape[0]

    fj = fij_ref[...]
    h = _ssp(jnp.dot(fj, we1_ref[...], preferred_element_type=jnp.float32)
             + be1_ref[...])
    rad = (jnp.dot(h, we2_ref[...], preferred_element_type=jnp.float32)
           + be2_ref[...])

    aux = aux_ref[...]
    dl = dm_ref[:, 0, :]
    ml = dm_ref[:, 1, :]
    cm = (dl + 1e-7 < _CUTOFF).astype(jnp.float32)
    dm = dl * (cm * (1.0 / _CUTOFF))
    cut = jnp.exp(1.0 - 1.0 / (1.0 - dm * dm)) * cm
    d2 = dl * dl
    cl = cut * ml / (d2 * d2 * dl)
    clt = cl.T
    rad = jnp.concatenate(
        [rad[i * 128:(i + 1) * 128] * clt[:, i:i + 1]
         for i in range(EB // 128)], axis=0)

    mui = mui_ref[...]
    r = jnp.zeros((EB, F), jnp.float32)
    for x in range(3):
        mix = mui[:, x * F:(x + 1) * F]
        mib = jnp.broadcast_to(
            mix[:, None, :], (TA, n_nbr, F)).reshape(EB, F)
        r = r + mib * aux[:, x:x + 1]

    v = ((t1_ref[...] - r) * rad).reshape(TA, n_nbr, F).sum(axis=1)

    v = _ssp(jnp.dot(v, w1_ref[...], preferred_element_type=jnp.float32)
             + b1_ref[...])
    out_ref[...] = (jnp.dot(v, w2_ref[...], preferred_element_type=jnp.float32)
                    + b2_ref[...])


def _tc_call(t1, mu_r, fij, aux, dm2, We1, be1, We2, be2, W1, b1, W2, b2,
             n_nbr):
    R, D = mu_r.shape
    F = D // 3
    G = fij.shape[1]
    AF = W2.shape[1]
    TA = 128
    EB = TA * n_nbr
    grid = (R // TA,)

    full = lambda a: pl.BlockSpec(a.shape, lambda i: (0, 0))
    return pl.pallas_call(
        functools.partial(_tc_body, n_nbr=n_nbr),
        grid=grid,
        in_specs=[
            pl.BlockSpec((EB, F), lambda i: (i, 0)),
            pl.BlockSpec((TA, D), lambda i: (i, 0)),
            pl.BlockSpec((EB, G), lambda i: (i, 0)),
            pl.BlockSpec((EB, 8), lambda i: (i, 0)),
            pl.BlockSpec((EB // 128, 2, 128), lambda i: (i, 0, 0)),
            full(We1), full(be1), full(We2), full(be2),
            full(W1), full(b1), full(W2), full(b2),
        ],
        out_specs=pl.BlockSpec((TA, AF), lambda i: (i, 0)),
        out_shape=jax.ShapeDtypeStruct((R, AF), jnp.float32),
    )(t1, mu_r, fij, aux, dm2, We1, be1, We2, be2, W1, b1, W2, b2)


def kernel(mu, distances, distance_vector, neighbors, f_ij, neighbor_mask,
           W1, b1, W2, b2, We1, be1, We2, be2):
    B, A, F, X = mu.shape
    N = distances.shape[-1]
    G = f_ij.shape[-1]
    E = B * A * N
    CW = 2 * N

    mu_r = mu.transpose(0, 1, 3, 2).reshape(B * A, X * F)
    idx = (neighbors.astype(jnp.int32)
           + (jnp.arange(B, dtype=jnp.int32) * A)[:, None, None])
    dvec = distance_vector.reshape(E, X).astype(jnp.float32)
    dist = distances.reshape(E, 1).astype(jnp.float32)
    aux_sc = jnp.concatenate([dvec, dist], axis=1)

    t1 = _sc_interact(mu_r, idx.reshape(E // CW, CW), aux_sc.reshape(E * 4), N)

    aux = jnp.concatenate([
        dvec, dist,
        neighbor_mask.reshape(E, 1).astype(jnp.float32),
        jnp.zeros((E, 3), jnp.float32),
    ], axis=1)
    dm2 = jnp.stack([distances.reshape(E // 128, 128).astype(jnp.float32),
                     neighbor_mask.reshape(E // 128, 128).astype(jnp.float32)],
                    axis=1)
    fij = f_ij.reshape(E, G)

    out = _tc_call(t1, mu_r, fij, aux, dm2,
                   We1, be1.reshape(1, -1), We2, be2.reshape(1, -1),
                   W1, b1.reshape(1, -1), W2, b2.reshape(1, -1), N)
    return out.reshape(B, A, -1)


# speedup vs baseline: 1.1697x; 1.1697x over previous
import functools

import jax
import jax.numpy as jnp
from jax import lax
from jax.experimental import pallas as pl
from jax.experimental.pallas import tpu as pltpu
from jax.experimental.pallas import tpu_sc as plsc

_CUTOFF = 5.0
_LOG2 = 0.6931471805599453


def _sc_interact(table, idx2d, aux, n_nbr):
    R, D = table.shape
    F = D // 3
    n_rows, CW = idx2d.shape
    E = n_rows * CW
    NW = 32
    per_w = E // NW
    n_ch = n_rows // NW
    NA = per_w // n_nbr
    A_CH = CW // n_nbr
    NV = D // 16
    KF = F // 16

    mesh = plsc.VectorSubcoreMesh(core_axis_name="c", subcore_axis_name="s")

    @functools.partial(
        pl.kernel,
        mesh=mesh,
        compiler_params=pltpu.CompilerParams(needs_layout_passes=False),
        out_type=jax.ShapeDtypeStruct((E, F), jnp.float32),
        scratch_types=[
            pltpu.VMEM((n_ch, CW), jnp.int32),
            pltpu.VMEM((NA, D), jnp.float32),
            pltpu.VMEM((per_w * 4,), jnp.float32),
            pltpu.VMEM((CW, D), jnp.float32),
            pltpu.VMEM((CW, D), jnp.float32),
            pltpu.VMEM((CW, F), jnp.float32),
            pltpu.VMEM((CW, F), jnp.float32),
            pltpu.SemaphoreType.DMA,
            pltpu.SemaphoreType.DMA,
            pltpu.SemaphoreType.DMA,
            pltpu.SemaphoreType.DMA,
        ],
    )
    def k(table_hbm, idx_hbm, aux_hbm, out_hbm, idx_v, mi_v, aux_v,
          mj0, mj1, tb0, tb1, g0, g1, p0, p1):
        wid = lax.axis_index("s") * 2 + lax.axis_index("c")
        pltpu.sync_copy(idx_hbm.at[pl.ds(wid * n_ch, n_ch)], idx_v)
        pltpu.sync_copy(table_hbm.at[pl.ds(wid * NA, NA)], mi_v)
        pltpu.sync_copy(aux_hbm.at[pl.ds(wid * per_w * 4, per_w * 4)], aux_v)
        ebase = wid * per_w

        def gather(c, mj, g):
            pltpu.async_copy(table_hbm.at[idx_v.at[c]], mj, g)

        def gwait(c, mj, g):
            pltpu.make_async_copy(table_hbm.at[idx_v.at[c]], mj, g).wait()

        def wb(c, tb, p):
            pltpu.async_copy(tb, out_hbm.at[pl.ds(ebase + c * CW, CW)], p)

        def wbwait(tb, p):
            pltpu.make_async_copy(tb, out_hbm.at[pl.ds(ebase, CW)], p).wait()

        def compute(c, mj, tb):
            for ai in range(A_CH):
                a_loc = c * A_CH + ai
                mi = [mi_v[a_loc, pl.ds(j * 16, 16)] for j in range(NV)]

                @plsc.parallel_loop(0, n_nbr, unroll=8)
                def nb_body(n):
                    e_loc = ai * n_nbr + n
                    eg = c * CW + e_loc
                    i0 = jnp.full((16,), eg * 4, jnp.int32)
                    ld = lambda col: plsc.load_gather(aux_v, [i0 + col])
                    dv0, dv1, dv2, db = ld(0), ld(1), ld(2), ld(3)
                    c3d2 = 3.0 * db * db
                    for kf in range(KF):
                        mja = mj[e_loc, pl.ds(kf * 16, 16)]
                        mjb = mj[e_loc, pl.ds(F + kf * 16, 16)]
                        mjc = mj[e_loc, pl.ds(2 * F + kf * 16, 16)]
                        pdd = (mi[kf] * mja + mi[KF + kf] * mjb
                               + mi[2 * KF + kf] * mjc)
                        qdd = dv0 * mja + dv1 * mjb + dv2 * mjc
                        tb[e_loc, pl.ds(kf * 16, 16)] = pdd * c3d2 - qdd

        gather(0, mj0, g0)

        def pair(i, _):
            c0 = 2 * i
            c1 = 2 * i + 1
            gather(c1, mj1, g1)
            gwait(c0, mj0, g0)

            @pl.when(i > 0)
            def _():
                wbwait(tb0, p0)

            compute(c0, mj0, tb0)
            wb(c0, tb0, p0)

            @pl.when(c0 + 2 < n_ch)
            def _():
                gather(c0 + 2, mj0, g0)

            gwait(c1, mj1, g1)

            @pl.when(i > 0)
            def _():
                wbwait(tb1, p1)

            compute(c1, mj1, tb1)
            wb(c1, tb1, p1)
            return 0

        lax.fori_loop(0, n_ch // 2, pair, 0)
        wbwait(tb0, p0)
        wbwait(tb1, p1)

    return k(table, idx2d, aux)


def _ssp(x):
    return jnp.maximum(x, 0.0) + jnp.log1p(jnp.exp(-jnp.abs(x))) - _LOG2


def _tc_body(t1_ref, mui_ref, fij_ref, aux_ref, dm_ref,
             we1_ref, be1_ref, we2_ref, be2_ref,
             w1_ref, b1_ref, w2_ref, b2_ref, out_ref, *, n_nbr):
    EB, F = t1_ref.shape
    TA = mui_ref.shape[0]

    fj = fij_ref[...]
    h = _ssp(jnp.dot(fj, we1_ref[...], preferred_element_type=jnp.float32)
             + be1_ref[...])
    rad = (jnp.dot(h, we2_ref[...], preferred_element_type=jnp.float32)
           + be2_ref[...])

    aux = aux_ref[...]
    dl = dm_ref[:, 0, :]
    ml = dm_ref[:, 1, :]
    cm = (dl + 1e-7 < _CUTOFF).astype(jnp.float32)
    dm = dl * (cm * (1.0 / _CUTOFF))
    cut = jnp.exp(1.0 - 1.0 / (1.0 - dm * dm)) * cm
    d2 = dl * dl
    cl = cut * ml / (d2 * d2 * dl)
    clt = cl.T
    rad = jnp.concatenate(
        [rad[i * 128:(i + 1) * 128] * clt[:, i:i + 1]
         for i in range(EB // 128)], axis=0)

    mui = mui_ref[...]
    r = jnp.zeros((EB, F), jnp.float32)
    for x in range(3):
        mix = mui[:, x * F:(x + 1) * F]
        mib = jnp.broadcast_to(
            mix[:, None, :], (TA, n_nbr, F)).reshape(EB, F)
        r = r + mib * aux[:, x:x + 1]

    v = ((t1_ref[...] - r) * rad).reshape(TA, n_nbr, F).sum(axis=1)

    v = _ssp(jnp.dot(v, w1_ref[...], preferred_element_type=jnp.float32)
             + b1_ref[...])
    out_ref[...] = (jnp.dot(v, w2_ref[...], preferred_element_type=jnp.float32)
                    + b2_ref[...])


def _tc_call(t1, mu_r, fij, aux, dm2, We1, be1, We2, be2, W1, b1, W2, b2,
             n_nbr):
    R, D = mu_r.shape
    F = D // 3
    G = fij.shape[1]
    AF = W2.shape[1]
    TA = 128
    EB = TA * n_nbr
    grid = (R // TA,)

    full = lambda a: pl.BlockSpec(a.shape, lambda i: (0, 0))
    return pl.pallas_call(
        functools.partial(_tc_body, n_nbr=n_nbr),
        grid=grid,
        in_specs=[
            pl.BlockSpec((EB, F), lambda i: (i, 0)),
            pl.BlockSpec((TA, D), lambda i: (i, 0)),
            pl.BlockSpec((EB, G), lambda i: (i, 0)),
            pl.BlockSpec((EB, 8), lambda i: (i, 0)),
            pl.BlockSpec((EB // 128, 2, 128), lambda i: (i, 0, 0)),
            full(We1), full(be1), full(We2), full(be2),
            full(W1), full(b1), full(W2), full(b2),
        ],
        out_specs=pl.BlockSpec((TA, AF), lambda i: (i, 0)),
        out_shape=jax.ShapeDtypeStruct((R, AF), jnp.float32),
    )(t1, mu_r, fij, aux, dm2, We1, be1, We2, be2, W1, b1, W2, b2)


def kernel(mu, distances, distance_vector, neighbors, f_ij, neighbor_mask,
           W1, b1, W2, b2, We1, be1, We2, be2):
    B, A, F, X = mu.shape
    N = distances.shape[-1]
    G = f_ij.shape[-1]
    E = B * A * N
    CW = 2 * N

    mu_r = mu.transpose(0, 1, 3, 2).reshape(B * A, X * F)
    idx = (neighbors.astype(jnp.int32)
           + (jnp.arange(B, dtype=jnp.int32) * A)[:, None, None])
    dvec = distance_vector.reshape(E, X).astype(jnp.float32)
    dist = distances.reshape(E, 1).astype(jnp.float32)
    aux_sc = jnp.concatenate([dvec, dist], axis=1)

    t1 = _sc_interact(mu_r, idx.reshape(E // CW, CW), aux_sc.reshape(E * 4), N)

    aux = jnp.concatenate([
        dvec, dist,
        neighbor_mask.reshape(E, 1).astype(jnp.float32),
        jnp.zeros((E, 3), jnp.float32),
    ], axis=1)
    dm2 = jnp.stack([distances.reshape(E // 128, 128).astype(jnp.float32),
                     neighbor_mask.reshape(E // 128, 128).astype(jnp.float32)],
                    axis=1)
    fij = f_ij.reshape(E, G)

    out = _tc_call(t1, mu_r, fij, aux, dm2,
                   We1, be1.reshape(1, -1), We2, be2.reshape(1, -1),
                   W1, b1.reshape(1, -1), W2, b2.reshape(1, -1), N)
    return out.reshape(B, A, -1)


# speedup vs baseline: 1.1868x; 1.0146x over previous
import functools

import jax
import jax.numpy as jnp
from jax import lax
from jax.experimental import pallas as pl
from jax.experimental.pallas import tpu as pltpu
from jax.experimental.pallas import tpu_sc as plsc

_CUTOFF = 5.0
_LOG2 = 0.6931471805599453


def _sc_interact(table, idx2d, dvec_flat, dist_flat, n_nbr, n_batch):
    R, D = table.shape
    F = D // 3
    n_rows, CW = idx2d.shape
    E = n_rows * CW
    NW = 32
    per_w = E // NW
    n_ch = n_rows // NW
    NA = per_w // n_nbr
    A_CH = CW // n_nbr
    NV = D // 16
    KF = F // 16

    mesh = plsc.VectorSubcoreMesh(core_axis_name="c", subcore_axis_name="s")

    @functools.partial(
        pl.kernel,
        mesh=mesh,
        compiler_params=pltpu.CompilerParams(needs_layout_passes=False),
        out_type=jax.ShapeDtypeStruct((E, F), jnp.float32),
        scratch_types=[
            pltpu.VMEM((n_ch, CW), jnp.int32),
            pltpu.VMEM((NA, D), jnp.float32),
            pltpu.VMEM((per_w * 3,), jnp.float32),
            pltpu.VMEM((per_w,), jnp.float32),
            pltpu.VMEM((CW, D), jnp.float32),
            pltpu.VMEM((CW, D), jnp.float32),
            pltpu.VMEM((CW, F), jnp.float32),
            pltpu.VMEM((CW, F), jnp.float32),
            pltpu.SemaphoreType.DMA,
            pltpu.SemaphoreType.DMA,
            pltpu.SemaphoreType.DMA,
            pltpu.SemaphoreType.DMA,
        ],
    )
    def k(table_hbm, idx_hbm, dvec_hbm, dist_hbm, out_hbm, idx_v, mi_v,
          dvec_v, dist_v, mj0, mj1, tb0, tb1, g0, g1, p0, p1):
        wid = lax.axis_index("s") * 2 + lax.axis_index("c")
        pltpu.sync_copy(idx_hbm.at[pl.ds(wid * n_ch, n_ch)], idx_v)
        pltpu.sync_copy(table_hbm.at[pl.ds(wid * NA, NA)], mi_v)
        pltpu.sync_copy(dvec_hbm.at[pl.ds(wid * per_w * 3, per_w * 3)], dvec_v)
        pltpu.sync_copy(dist_hbm.at[pl.ds(wid * per_w, per_w)], dist_v)
        ebase = wid * per_w

        row_off = jnp.full((16,), (R // n_batch), jnp.int32) * (
            wid // (NW // n_batch))
        for rr in range(n_ch):
            for jj in range(CW // 16):
                sl = pl.ds(jj * 16, 16)
                idx_v[rr, sl] = idx_v[rr, sl] + row_off

        def gather(c, mj, g):
            pltpu.async_copy(table_hbm.at[idx_v.at[c]], mj, g)

        def gwait(c, mj, g):
            pltpu.make_async_copy(table_hbm.at[idx_v.at[c]], mj, g).wait()

        def wb(c, tb, p):
            pltpu.async_copy(tb, out_hbm.at[pl.ds(ebase + c * CW, CW)], p)

        def wbwait(tb, p):
            pltpu.make_async_copy(tb, out_hbm.at[pl.ds(ebase, CW)], p).wait()

        def compute(c, mj, tb):
            for ai in range(A_CH):
                a_loc = c * A_CH + ai
                mi = [mi_v[a_loc, pl.ds(j * 16, 16)] for j in range(NV)]

                @plsc.parallel_loop(0, n_nbr, unroll=8)
                def nb_body(n):
                    e_loc = ai * n_nbr + n
                    eg = c * CW + e_loc
                    i0 = jnp.full((16,), eg * 3, jnp.int32)
                    ld = lambda col: plsc.load_gather(dvec_v, [i0 + col])
                    dv0, dv1, dv2 = ld(0), ld(1), ld(2)
                    db = plsc.load_gather(
                        dist_v, [jnp.full((16,), eg, jnp.int32)])
                    c3d2 = 3.0 * db * db
                    for kf in range(KF):
                        mja = mj[e_loc, pl.ds(kf * 16, 16)]
                        mjb = mj[e_loc, pl.ds(F + kf * 16, 16)]
                        mjc = mj[e_loc, pl.ds(2 * F + kf * 16, 16)]
                        pdd = (mi[kf] * mja + mi[KF + kf] * mjb
                               + mi[2 * KF + kf] * mjc)
                        qdd = dv0 * mja + dv1 * mjb + dv2 * mjc
                        tb[e_loc, pl.ds(kf * 16, 16)] = pdd * c3d2 - qdd

        gather(0, mj0, g0)

        def pair(i, _):
            c0 = 2 * i
            c1 = 2 * i + 1
            gather(c1, mj1, g1)
            gwait(c0, mj0, g0)

            @pl.when(i > 0)
            def _():
                wbwait(tb0, p0)

            compute(c0, mj0, tb0)
            wb(c0, tb0, p0)

            @pl.when(c0 + 2 < n_ch)
            def _():
                gather(c0 + 2, mj0, g0)

            gwait(c1, mj1, g1)

            @pl.when(i > 0)
            def _():
                wbwait(tb1, p1)

            compute(c1, mj1, tb1)
            wb(c1, tb1, p1)
            return 0

        lax.fori_loop(0, n_ch // 2, pair, 0)
        wbwait(tb0, p0)
        wbwait(tb1, p1)

    return k(table, idx2d, dvec_flat, dist_flat)


def _ssp(x):
    return jnp.maximum(x, 0.0) + jnp.log1p(jnp.exp(-jnp.abs(x))) - _LOG2


def _tc_body(t1_ref, mui_ref, fij_ref, dvec_ref, d_ref, m_ref,
             we1_ref, be1_ref, we2_ref, be2_ref,
             w1_ref, b1_ref, w2_ref, b2_ref, out_ref, *, n_nbr):
    EB, F = t1_ref.shape
    TA = mui_ref.shape[0]

    fj = fij_ref[...]
    h = _ssp(jnp.dot(fj, we1_ref[...], preferred_element_type=jnp.float32)
             + be1_ref[...])
    rad = (jnp.dot(h, we2_ref[...], preferred_element_type=jnp.float32)
           + be2_ref[...])

    dvec = dvec_ref[...]
    dl = d_ref[...]
    ml = m_ref[...]
    cm = (dl + 1e-7 < _CUTOFF).astype(jnp.float32)
    dm = dl * (cm * (1.0 / _CUTOFF))
    cut = jnp.exp(1.0 - 1.0 / (1.0 - dm * dm)) * cm
    d2 = dl * dl
    cl = cut * ml / (d2 * d2 * dl)
    clt = cl.T
    rad = jnp.concatenate(
        [rad[i * 128:(i + 1) * 128] * clt[:, i:i + 1]
         for i in range(EB // 128)], axis=0)

    mui = mui_ref[...]
    r = jnp.zeros((EB, F), jnp.float32)
    for x in range(3):
        mix = mui[:, x * F:(x + 1) * F]
        mib = jnp.broadcast_to(
            mix[:, None, :], (TA, n_nbr, F)).reshape(EB, F)
        r = r + mib * dvec[:, x:x + 1]

    v = ((t1_ref[...] - r) * rad).reshape(TA, n_nbr, F).sum(axis=1)

    v = _ssp(jnp.dot(v, w1_ref[...], preferred_element_type=jnp.float32)
             + b1_ref[...])
    out_ref[...] = (jnp.dot(v, w2_ref[...], preferred_element_type=jnp.float32)
                    + b2_ref[...])


def _tc_call(t1, mu_r, fij, dvec, d128, m128, We1, be1, We2, be2,
             W1, b1, W2, b2, n_nbr):
    R, D = mu_r.shape
    F = D // 3
    G = fij.shape[1]
    AF = W2.shape[1]
    TA = 256
    EB = TA * n_nbr
    grid = (R // TA,)

    full = lambda a: pl.BlockSpec(a.shape, lambda i: (0, 0))
    return pl.pallas_call(
        functools.partial(_tc_body, n_nbr=n_nbr),
        grid=grid,
        in_specs=[
            pl.BlockSpec((EB, F), lambda i: (i, 0)),
            pl.BlockSpec((TA, D), lambda i: (i, 0)),
            pl.BlockSpec((EB, G), lambda i: (i, 0)),
            pl.BlockSpec((EB, 3), lambda i: (i, 0)),
            pl.BlockSpec((EB // 128, 128), lambda i: (i, 0)),
            pl.BlockSpec((EB // 128, 128), lambda i: (i, 0)),
            full(We1), full(be1), full(We2), full(be2),
            full(W1), full(b1), full(W2), full(b2),
        ],
        out_specs=pl.BlockSpec((TA, AF), lambda i: (i, 0)),
        out_shape=jax.ShapeDtypeStruct((R, AF), jnp.float32),
    )(t1, mu_r, fij, dvec, d128, m128, We1, be1, We2, be2, W1, b1, W2, b2)


def kernel(mu, distances, distance_vector, neighbors, f_ij, neighbor_mask,
           W1, b1, W2, b2, We1, be1, We2, be2):
    B, A, F, X = mu.shape
    N = distances.shape[-1]
    G = f_ij.shape[-1]
    E = B * A * N
    CW = 2 * N

    mu_r = mu.transpose(0, 1, 3, 2).reshape(B * A, X * F)
    idx = neighbors.astype(jnp.int32)
    dvec = distance_vector.reshape(E, X).astype(jnp.float32)

    t1 = _sc_interact(mu_r, idx.reshape(E // CW, CW),
                      dvec.reshape(E * X), distances.reshape(E), N, B)

    d128 = distances.reshape(E // 128, 128)
    m128 = neighbor_mask.reshape(E // 128, 128).astype(jnp.float32)
    fij = f_ij.reshape(E, G)

    out = _tc_call(t1, mu_r, fij, dvec, d128, m128,
                   We1, be1.reshape(1, -1), We2, be2.reshape(1, -1),
                   W1, b1.reshape(1, -1), W2, b2.reshape(1, -1), N)
    return out.reshape(B, A, -1)


# speedup vs baseline: 1.2121x; 1.0213x over previous
import functools

import jax
import jax.numpy as jnp
from jax import lax
from jax.experimental import pallas as pl
from jax.experimental.pallas import tpu as pltpu
from jax.experimental.pallas import tpu_sc as plsc

_CUTOFF = 5.0
_LOG2 = 0.6931471805599453


def _sc_interact(table, idx2d, dvec_flat, dist_flat, n_nbr, n_batch):
    R, D = table.shape
    F = D // 3
    n_rows, CW = idx2d.shape
    E = n_rows * CW
    NW = 32
    per_w = E // NW
    n_ch = n_rows // NW
    NA = per_w // n_nbr
    A_CH = CW // n_nbr
    NV = D // 16
    KF = F // 16

    mesh = plsc.VectorSubcoreMesh(core_axis_name="c", subcore_axis_name="s")

    @functools.partial(
        pl.kernel,
        mesh=mesh,
        compiler_params=pltpu.CompilerParams(needs_layout_passes=False),
        out_type=jax.ShapeDtypeStruct((E, F), jnp.float32),
        scratch_types=[
            pltpu.VMEM((n_ch, CW), jnp.int32),
            pltpu.VMEM((NA, D), jnp.float32),
            pltpu.VMEM((per_w * 3,), jnp.float32),
            pltpu.VMEM((per_w,), jnp.float32),
            pltpu.VMEM((CW, D), jnp.float32),
            pltpu.VMEM((CW, D), jnp.float32),
            pltpu.VMEM((CW, F), jnp.float32),
            pltpu.VMEM((CW, F), jnp.float32),
            pltpu.SemaphoreType.DMA,
            pltpu.SemaphoreType.DMA,
            pltpu.SemaphoreType.DMA,
            pltpu.SemaphoreType.DMA,
        ],
    )
    def k(table_hbm, idx_hbm, dvec_hbm, dist_hbm, out_hbm, idx_v, mi_v,
          dvec_v, dist_v, mj0, mj1, tb0, tb1, g0, g1, p0, p1):
        wid = lax.axis_index("s") * 2 + lax.axis_index("c")
        pltpu.sync_copy(idx_hbm.at[pl.ds(wid * n_ch, n_ch)], idx_v)
        pltpu.sync_copy(table_hbm.at[pl.ds(wid * NA, NA)], mi_v)
        pltpu.sync_copy(dvec_hbm.at[pl.ds(wid * per_w * 3, per_w * 3)], dvec_v)
        pltpu.sync_copy(dist_hbm.at[pl.ds(wid * per_w, per_w)], dist_v)
        ebase = wid * per_w

        row_off = jnp.full((16,), (R // n_batch), jnp.int32) * (
            wid // (NW // n_batch))
        for rr in range(n_ch):
            for jj in range(CW // 16):
                sl = pl.ds(jj * 16, 16)
                idx_v[rr, sl] = idx_v[rr, sl] + row_off

        def gather(c, mj, g):
            pltpu.async_copy(table_hbm.at[idx_v.at[c]], mj, g)

        def gwait(c, mj, g):
            pltpu.make_async_copy(table_hbm.at[idx_v.at[c]], mj, g).wait()

        def wb(c, tb, p):
            pltpu.async_copy(tb, out_hbm.at[pl.ds(ebase + c * CW, CW)], p)

        def wbwait(tb, p):
            pltpu.make_async_copy(tb, out_hbm.at[pl.ds(ebase, CW)], p).wait()

        def compute(c, mj, tb):
            for ai in range(A_CH):
                a_loc = c * A_CH + ai
                mi = [mi_v[a_loc, pl.ds(j * 16, 16)] for j in range(NV)]

                @plsc.parallel_loop(0, n_nbr, unroll=8)
                def nb_body(n):
                    e_loc = ai * n_nbr + n
                    eg = c * CW + e_loc
                    i0 = jnp.full((16,), eg * 3, jnp.int32)
                    ld = lambda col: plsc.load_gather(dvec_v, [i0 + col])
                    dv0, dv1, dv2 = ld(0), ld(1), ld(2)
                    db = plsc.load_gather(
                        dist_v, [jnp.full((16,), eg, jnp.int32)])
                    c3d2 = 3.0 * db * db
                    for kf in range(KF):
                        mja = mj[e_loc, pl.ds(kf * 16, 16)]
                        mjb = mj[e_loc, pl.ds(F + kf * 16, 16)]
                        mjc = mj[e_loc, pl.ds(2 * F + kf * 16, 16)]
                        pdd = (mi[kf] * mja + mi[KF + kf] * mjb
                               + mi[2 * KF + kf] * mjc)
                        qdd = dv0 * mja + dv1 * mjb + dv2 * mjc
                        tb[e_loc, pl.ds(kf * 16, 16)] = pdd * c3d2 - qdd

        gather(0, mj0, g0)

        def pair(i, _):
            c0 = 2 * i
            c1 = 2 * i + 1
            gather(c1, mj1, g1)
            gwait(c0, mj0, g0)

            @pl.when(i > 0)
            def _():
                wbwait(tb0, p0)

            compute(c0, mj0, tb0)
            wb(c0, tb0, p0)

            @pl.when(c0 + 2 < n_ch)
            def _():
                gather(c0 + 2, mj0, g0)

            gwait(c1, mj1, g1)

            @pl.when(i > 0)
            def _():
                wbwait(tb1, p1)

            compute(c1, mj1, tb1)
            wb(c1, tb1, p1)
            return 0

        lax.fori_loop(0, n_ch // 2, pair, 0)
        wbwait(tb0, p0)
        wbwait(tb1, p1)

    return k(table, idx2d, dvec_flat, dist_flat)


def _ssp(x):
    return jnp.maximum(x, 0.0) + jnp.log1p(jnp.exp(-jnp.abs(x))) - _LOG2


def _rad_body(fij_ref, d_ref, m_ref, we1_ref, be1_ref, we2_ref, be2_ref,
              rad_ref):
    EB, F = rad_ref.shape
    fj = fij_ref[...]
    h = _ssp(jnp.dot(fj, we1_ref[...], preferred_element_type=jnp.float32)
             + be1_ref[...])
    rad = (jnp.dot(h, we2_ref[...], preferred_element_type=jnp.float32)
           + be2_ref[...])

    dl = d_ref[...]
    ml = m_ref[...]
    cm = (dl + 1e-7 < _CUTOFF).astype(jnp.float32)
    dm = dl * (cm * (1.0 / _CUTOFF))
    cut = jnp.exp(1.0 - 1.0 / (1.0 - dm * dm)) * cm
    d2 = dl * dl
    cl = cut * ml / (d2 * d2 * dl)
    clt = cl.T
    rad_ref[...] = jnp.concatenate(
        [rad[i * 128:(i + 1) * 128] * clt[:, i:i + 1]
         for i in range(EB // 128)], axis=0)


def _rad_call(fij, d128, m128, We1, be1, We2, be2, n_nbr, F):
    E, G = fij.shape
    TA = 128
    EB = TA * n_nbr
    grid = (E // EB,)
    full = lambda a: pl.BlockSpec(a.shape, lambda i: (0, 0))
    return pl.pallas_call(
        _rad_body,
        grid=grid,
        in_specs=[
            pl.BlockSpec((EB, G), lambda i: (i, 0)),
            pl.BlockSpec((EB // 128, 128), lambda i: (i, 0)),
            pl.BlockSpec((EB // 128, 128), lambda i: (i, 0)),
            full(We1), full(be1), full(We2), full(be2),
        ],
        out_specs=pl.BlockSpec((EB, F), lambda i: (i, 0)),
        out_shape=jax.ShapeDtypeStruct((E, F), jnp.float32),
    )(fij, d128, m128, We1, be1, We2, be2)


def _tc_body(t1_ref, rad_ref, mui_ref, dvec_ref,
             w1_ref, b1_ref, w2_ref, b2_ref, out_ref, *, n_nbr):
    EB, F = t1_ref.shape
    TA = mui_ref.shape[0]

    dvec = dvec_ref[...]
    mui = mui_ref[...]
    r = jnp.zeros((EB, F), jnp.float32)
    for x in range(3):
        mix = mui[:, x * F:(x + 1) * F]
        mib = jnp.broadcast_to(
            mix[:, None, :], (TA, n_nbr, F)).reshape(EB, F)
        r = r + mib * dvec[:, x:x + 1]

    v = ((t1_ref[...] - r) * rad_ref[...]).reshape(TA, n_nbr, F).sum(axis=1)

    v = _ssp(jnp.dot(v, w1_ref[...], preferred_element_type=jnp.float32)
             + b1_ref[...])
    out_ref[...] = (jnp.dot(v, w2_ref[...], preferred_element_type=jnp.float32)
                    + b2_ref[...])


def _tc_call(t1, rad, mu_r, dvec, W1, b1, W2, b2, n_nbr):
    R, D = mu_r.shape
    F = D // 3
    AF = W2.shape[1]
    TA = 128
    EB = TA * n_nbr
    grid = (R // TA,)

    full = lambda a: pl.BlockSpec(a.shape, lambda i: (0, 0))
    return pl.pallas_call(
        functools.partial(_tc_body, n_nbr=n_nbr),
        grid=grid,
        in_specs=[
            pl.BlockSpec((EB, F), lambda i: (i, 0)),
            pl.BlockSpec((EB, F), lambda i: (i, 0)),
            pl.BlockSpec((TA, D), lambda i: (i, 0)),
            pl.BlockSpec((EB, 3), lambda i: (i, 0)),
            full(W1), full(b1), full(W2), full(b2),
        ],
        out_specs=pl.BlockSpec((TA, AF), lambda i: (i, 0)),
        out_shape=jax.ShapeDtypeStruct((R, AF), jnp.float32),
    )(t1, rad, mu_r, dvec, W1, b1, W2, b2)


def kernel(mu, distances, distance_vector, neighbors, f_ij, neighbor_mask,
           W1, b1, W2, b2, We1, be1, We2, be2):
    B, A, F, X = mu.shape
    N = distances.shape[-1]
    G = f_ij.shape[-1]
    E = B * A * N
    CW = 2 * N

    mu_r = mu.transpose(0, 1, 3, 2).reshape(B * A, X * F)
    idx = neighbors.astype(jnp.int32)
    dvec = distance_vector.reshape(E, X).astype(jnp.float32)

    t1 = _sc_interact(mu_r, idx.reshape(E // CW, CW),
                      dvec.reshape(E * X), distances.reshape(E), N, B)

    d128 = distances.reshape(E // 128, 128)
    m128 = neighbor_mask.reshape(E // 128, 128).astype(jnp.float32)
    fij = f_ij.reshape(E, G)

    rad = _rad_call(fij, d128, m128, We1, be1.reshape(1, -1),
                    We2, be2.reshape(1, -1), N, F)

    out = _tc_call(t1, rad, mu_r, dvec,
                   W1, b1.reshape(1, -1), W2, b2.reshape(1, -1), N)
    return out.reshape(B, A, -1)
